# baseline (speedup 1.0000x reference)
import functools

import jax
import jax.numpy as jnp
from jax import lax
from jax.experimental import pallas as pl
from jax.experimental.pallas import tpu as pltpu
from jax.experimental.pallas import tpu_sc as plsc

N = 10000
E = 320000
D = 128

NC = 2
NS = 16
NW = NC * NS
SUB = 64
NROW = E // SUB
TPT = 160
PAD_ROWS = TPT * NW
TLOOP = 162
GR = 8
ROWS_PER_TILE = 624
CP = 104
TAIL_R0 = NS * ROWS_PER_TILE
TAIL_ROWS = N - TAIL_R0
NLANE = D // 16


def _sc_aggregate(node_hbm, edge_hbm, src_hbm, dst_hbm, out_hbm,
                  shared_agg, sdb, ddb,
                  bg0, bg1, bg2, be0, be1,
                  sem_i, sg0, sg1, sg2, se0, se1, ss0, ss1, ss2):
    c = lax.axis_index("c")
    s = lax.axis_index("s")
    wid = s * NC + c
    row_base = wid * TPT

    pltpu.sync_copy(src_hbm.at[pl.ds(row_base, GR)], sdb.at[0])
    pltpu.sync_copy(dst_hbm.at[pl.ds(row_base, GR)], ddb.at[0])
    pltpu.async_copy(src_hbm.at[pl.ds(row_base + GR, GR)], sdb.at[1], sem_i)
    pltpu.async_copy(dst_hbm.at[pl.ds(row_base + GR, GR)], ddb.at[1], sem_i)

    row0 = s * ROWS_PER_TILE
    is_tail = s == NS - 1

    @pl.when(c == 0)
    def _():
        for k in range(ROWS_PER_TILE // CP):
            r0 = row0 + k * CP
            pltpu.sync_copy(node_hbm.at[pl.ds(r0, CP)],
                            shared_agg.at[pl.ds(r0, CP)])

        @pl.when(is_tail)
        def _():
            pltpu.sync_copy(node_hbm.at[pl.ds(TAIL_R0, TAIL_ROWS)],
                            shared_agg.at[pl.ds(TAIL_R0, TAIL_ROWS)])

    @pl.when(c != 0)
    def _():
        def zrow(r, carry):
            for j in range(NLANE):
                bg0[r, pl.ds(j * 16, 16)] = jnp.zeros((16,), jnp.float32)
            return carry
        lax.fori_loop(0, SUB, zrow, 0)
        for k in range(ROWS_PER_TILE // CP):
            r0 = row0 + k * CP
            for b in range(0, CP, SUB):
                nrow = min(SUB, CP - b)
                pltpu.sync_copy(bg0.at[pl.ds(0, nrow)],
                                shared_agg.at[pl.ds(r0 + b, nrow)])

        @pl.when(is_tail)
        def _():
            pltpu.sync_copy(bg0.at[pl.ds(0, TAIL_ROWS)],
                            shared_agg.at[pl.ds(TAIL_R0, TAIL_ROWS)])

    plsc.subcore_barrier()

    bgs = (bg0, bg1, bg2)
    bes = (be0, be1)
    sgs = (sg0, sg1, sg2)
    ses = (se0, se1)
    sss = (ss0, ss1, ss2)

    def idx_slice(buf, t):
        return buf.at[(t // GR) % 2, t % GR]

    def issue_loads(t, bg, sg, be, se):
        pltpu.async_copy(node_hbm.at[idx_slice(sdb, t)], bg, sg)
        grow = row_base + t
        eoff = jnp.minimum(grow, NROW - 1) * SUB
        pltpu.async_copy(edge_hbm.at[pl.ds(eoff, SUB)], be, se)

    issue_loads(jnp.int32(0), bg0, sg0, be0, se0)
    issue_loads(jnp.int32(1), bg1, sg1, be1, se1)

    def six_body(u, carry):
        for k in range(6):
            r3 = k % 3
            p2 = k % 2
            bg, sg = bgs[r3], sgs[r3]
            be, se = bes[p2], ses[p2]
            t = 6 * u + k
            grow = row_base + t
            pltpu.make_async_copy(
                node_hbm.at[idx_slice(sdb, t)], bg, sg).wait()
            eoff = jnp.minimum(grow, NROW - 1) * SUB
            pltpu.make_async_copy(
                edge_hbm.at[pl.ds(eoff, SUB)], be, se).wait()

            is_pad = jnp.logical_or(t >= TPT, grow >= NROW)

            @pl.when(jnp.logical_not(is_pad))
            def _():
                def rbody(r, rc):
                    for j in range(NLANE):
                        sl = pl.ds(j * 16, 16)
                        bg[r, sl] = jnp.maximum(bg[r, sl] + be[r, sl], 0.0)
                    return rc
                lax.fori_loop(0, SUB, rbody, 0)

            @pl.when(is_pad)
            def _():
                def zbody(r, rc):
                    for j in range(NLANE):
                        bg[r, pl.ds(j * 16, 16)] = jnp.zeros((16,),
                                                             jnp.float32)
                    return rc
                lax.fori_loop(0, SUB, zbody, 0)

            pltpu.async_copy(bg, shared_agg.at[idx_slice(ddb, t)],
                             sss[r3], add=True)

            @pl.when(t >= 1)
            def _():
                tp = t - 1
                pltpu.make_async_copy(
                    bgs[(k + 2) % 3], shared_agg.at[idx_slice(ddb, tp)],
                    sss[(k + 2) % 3]).wait()

            @pl.when(jnp.logical_and(t % GR == 0, jnp.logical_and(
                t >= GR, t < TLOOP - GR)))
            def _():
                g1 = t // GR + 1
                goff = jnp.minimum(row_base + GR * g1, PAD_ROWS - GR)
                pltpu.async_copy(src_hbm.at[pl.ds(goff, GR)],
                                 sdb.at[g1 % 2], sem_i)
                pltpu.async_copy(dst_hbm.at[pl.ds(goff, GR)],
                                 ddb.at[g1 % 2], sem_i)

            @pl.when(jnp.logical_and(t % GR == 6, t < TLOOP - 2))
            def _():
                g1 = t // GR + 1
                goff = jnp.minimum(row_base + GR * g1, PAD_ROWS - GR)
                pltpu.make_async_copy(
                    src_hbm.at[pl.ds(goff, GR)], sdb.at[g1 % 2],
                    sem_i).wait()
                pltpu.make_async_copy(
                    dst_hbm.at[pl.ds(goff, GR)], ddb.at[g1 % 2],
                    sem_i).wait()

            @pl.when(t + 2 < TLOOP)
            def _():
                issue_loads(t + 2, bgs[(k + 2) % 3], sgs[(k + 2) % 3],
                            be, se)
        return carry

    lax.fori_loop(0, TLOOP // 6, six_body, 0)

    tl = jnp.int32(TLOOP - 1)
    pltpu.make_async_copy(
        bgs[(TLOOP - 1) % 3], shared_agg.at[idx_slice(ddb, tl)],
        sss[(TLOOP - 1) % 3]).wait()

    plsc.subcore_barrier()

    for k in range(ROWS_PER_TILE // CP):
        r0 = row0 + k * CP
        pltpu.sync_copy(shared_agg.at[pl.ds(r0, CP)],
                        out_hbm.at[c, pl.ds(r0, CP)])

    @pl.when(is_tail)
    def _():
        pltpu.sync_copy(shared_agg.at[pl.ds(TAIL_R0, TAIL_ROWS)],
                        out_hbm.at[c, pl.ds(TAIL_R0, TAIL_ROWS)])


_sc_call = functools.partial(
    pl.kernel,
    out_type=jax.ShapeDtypeStruct((NC, N, D), jnp.float32),
    mesh=plsc.VectorSubcoreMesh(core_axis_name="c", subcore_axis_name="s"),
    scratch_types=[
        pltpu.VMEM_SHARED((N, D), jnp.float32),
        pltpu.VMEM((2, GR, SUB), jnp.int32),
        pltpu.VMEM((2, GR, SUB), jnp.int32),
        pltpu.VMEM((SUB, D), jnp.float32),
        pltpu.VMEM((SUB, D), jnp.float32),
        pltpu.VMEM((SUB, D), jnp.float32),
        pltpu.VMEM((SUB, D), jnp.float32),
        pltpu.VMEM((SUB, D), jnp.float32),
        pltpu.SemaphoreType.DMA,
        pltpu.SemaphoreType.DMA,
        pltpu.SemaphoreType.DMA,
        pltpu.SemaphoreType.DMA,
        pltpu.SemaphoreType.DMA,
        pltpu.SemaphoreType.DMA,
        pltpu.SemaphoreType.DMA,
        pltpu.SemaphoreType.DMA,
        pltpu.SemaphoreType.DMA,
    ],
)(_sc_aggregate)


def _mlp_bn(agg_ref, W1_ref, b1_ref, W2_ref, b2_ref, gamma_ref, beta_ref,
            out_ref):
    h0 = agg_ref[0] + agg_ref[1]
    h1 = jnp.maximum(
        jnp.dot(h0, W1_ref[...], preferred_element_type=jnp.float32)
        + b1_ref[...], 0.0)
    h2 = (jnp.dot(h1, W2_ref[...], preferred_element_type=jnp.float32)
          + b2_ref[...])
    mean = jnp.mean(h2, axis=0, keepdims=True)
    var = jnp.mean(h2 * h2, axis=0, keepdims=True) - mean * mean
    inv = jax.lax.rsqrt(var + 1e-5)
    out_ref[...] = (h2 - mean) * inv * gamma_ref[...] + beta_ref[...]


@jax.jit
def kernel(node_feats, edge_feats, W1, b1, W2, b2, gamma, beta, edge_index):
    pad = PAD_ROWS * SUB - E
    src = jnp.pad(edge_index[0], (0, pad)).reshape(PAD_ROWS, SUB)
    dst = jnp.pad(edge_index[1], (0, pad)).reshape(PAD_ROWS, SUB)
    agg = _sc_call(node_feats, edge_feats, src, dst)
    out = pl.pallas_call(
        _mlp_bn,
        out_shape=jax.ShapeDtypeStruct((N, D), jnp.float32),
    )(agg, W1, b1.reshape(1, D), W2, b2.reshape(1, D),
      gamma.reshape(1, D), beta.reshape(1, D))
    return out


# speedup vs baseline: 1.0045x; 1.0045x over previous
import functools

import jax
import jax.numpy as jnp
from jax import lax
from jax.experimental import pallas as pl
from jax.experimental.pallas import tpu as pltpu
from jax.experimental.pallas import tpu_sc as plsc

N = 10000
E = 320000
D = 128

NC = 2
NS = 16
NW = NC * NS
SUB = 32
NROW = E // SUB
TPT = 320
PAD_ROWS = TPT * NW
TLOOP = 324
GR = 8
ROWS_PER_TILE = 624
CP = 104
TAIL_R0 = NS * ROWS_PER_TILE
TAIL_ROWS = N - TAIL_R0
NLANE = D // 16


def _sc_aggregate(node_hbm, edge_hbm, src_hbm, dst_hbm, out_hbm,
                  shared_agg, sdb, ddb,
                  bg0, bg1, bg2, be0, be1,
                  sem_i, sg0, sg1, sg2, se0, se1, ss0, ss1, ss2):
    c = lax.axis_index("c")
    s = lax.axis_index("s")
    wid = s * NC + c
    row_base = wid * TPT

    pltpu.sync_copy(src_hbm.at[pl.ds(row_base, GR)], sdb.at[0])
    pltpu.sync_copy(dst_hbm.at[pl.ds(row_base, GR)], ddb.at[0])
    pltpu.async_copy(src_hbm.at[pl.ds(row_base + GR, GR)], sdb.at[1], sem_i)
    pltpu.async_copy(dst_hbm.at[pl.ds(row_base + GR, GR)], ddb.at[1], sem_i)

    row0 = s * ROWS_PER_TILE
    is_tail = s == NS - 1

    @pl.when(c == 0)
    def _():
        for k in range(ROWS_PER_TILE // CP):
            r0 = row0 + k * CP
            pltpu.sync_copy(node_hbm.at[pl.ds(r0, CP)],
                            shared_agg.at[pl.ds(r0, CP)])

        @pl.when(is_tail)
        def _():
            pltpu.sync_copy(node_hbm.at[pl.ds(TAIL_R0, TAIL_ROWS)],
                            shared_agg.at[pl.ds(TAIL_R0, TAIL_ROWS)])

    @pl.when(c != 0)
    def _():
        def zrow(r, carry):
            for j in range(NLANE):
                bg0[r, pl.ds(j * 16, 16)] = jnp.zeros((16,), jnp.float32)
            return carry
        lax.fori_loop(0, SUB, zrow, 0)
        for k in range(ROWS_PER_TILE // CP):
            r0 = row0 + k * CP
            for b in range(0, CP, SUB):
                nrow = min(SUB, CP - b)
                pltpu.sync_copy(bg0.at[pl.ds(0, nrow)],
                                shared_agg.at[pl.ds(r0 + b, nrow)])

        @pl.when(is_tail)
        def _():
            pltpu.sync_copy(bg0.at[pl.ds(0, TAIL_ROWS)],
                            shared_agg.at[pl.ds(TAIL_R0, TAIL_ROWS)])

    plsc.subcore_barrier()

    bgs = (bg0, bg1, bg2)
    bes = (be0, be1)
    sgs = (sg0, sg1, sg2)
    ses = (se0, se1)
    sss = (ss0, ss1, ss2)

    def idx_slice(buf, t):
        return buf.at[(t // GR) % 2, t % GR]

    def issue_loads(t, bg, sg, be, se):
        pltpu.async_copy(node_hbm.at[idx_slice(sdb, t)], bg, sg)
        grow = row_base + t
        eoff = jnp.minimum(grow, NROW - 1) * SUB
        pltpu.async_copy(edge_hbm.at[pl.ds(eoff, SUB)], be, se)

    issue_loads(jnp.int32(0), bg0, sg0, be0, se0)
    issue_loads(jnp.int32(1), bg1, sg1, be1, se1)

    def six_body(u, carry):
        for k in range(6):
            r3 = k % 3
            p2 = k % 2
            bg, sg = bgs[r3], sgs[r3]
            be, se = bes[p2], ses[p2]
            t = 6 * u + k
            grow = row_base + t
            pltpu.make_async_copy(
                node_hbm.at[idx_slice(sdb, t)], bg, sg).wait()
            eoff = jnp.minimum(grow, NROW - 1) * SUB
            pltpu.make_async_copy(
                edge_hbm.at[pl.ds(eoff, SUB)], be, se).wait()

            is_pad = jnp.logical_or(t >= TPT, grow >= NROW)

            @pl.when(jnp.logical_not(is_pad))
            def _():
                def rbody(r, rc):
                    for j in range(NLANE):
                        sl = pl.ds(j * 16, 16)
                        bg[r, sl] = jnp.maximum(bg[r, sl] + be[r, sl], 0.0)
                    return rc
                lax.fori_loop(0, SUB, rbody, 0)

            @pl.when(is_pad)
            def _():
                def zbody(r, rc):
                    for j in range(NLANE):
                        bg[r, pl.ds(j * 16, 16)] = jnp.zeros((16,),
                                                             jnp.float32)
                    return rc
                lax.fori_loop(0, SUB, zbody, 0)

            pltpu.async_copy(bg, shared_agg.at[idx_slice(ddb, t)],
                             sss[r3], add=True)

            @pl.when(t >= 1)
            def _():
                tp = t - 1
                pltpu.make_async_copy(
                    bgs[(k + 2) % 3], shared_agg.at[idx_slice(ddb, tp)],
                    sss[(k + 2) % 3]).wait()

            @pl.when(jnp.logical_and(t % GR == 0, jnp.logical_and(
                t >= GR, t < TLOOP - GR)))
            def _():
                g1 = t // GR + 1
                goff = jnp.minimum(row_base + GR * g1, PAD_ROWS - GR)
                pltpu.async_copy(src_hbm.at[pl.ds(goff, GR)],
                                 sdb.at[g1 % 2], sem_i)
                pltpu.async_copy(dst_hbm.at[pl.ds(goff, GR)],
                                 ddb.at[g1 % 2], sem_i)

            @pl.when(jnp.logical_and(t % GR == 6, t < TLOOP - 2))
            def _():
                g1 = t // GR + 1
                goff = jnp.minimum(row_base + GR * g1, PAD_ROWS - GR)
                pltpu.make_async_copy(
                    src_hbm.at[pl.ds(goff, GR)], sdb.at[g1 % 2],
                    sem_i).wait()
                pltpu.make_async_copy(
                    dst_hbm.at[pl.ds(goff, GR)], ddb.at[g1 % 2],
                    sem_i).wait()

            @pl.when(t + 2 < TLOOP)
            def _():
                issue_loads(t + 2, bgs[(k + 2) % 3], sgs[(k + 2) % 3],
                            be, se)
        return carry

    lax.fori_loop(0, TLOOP // 6, six_body, 0)

    tl = jnp.int32(TLOOP - 1)
    pltpu.make_async_copy(
        bgs[(TLOOP - 1) % 3], shared_agg.at[idx_slice(ddb, tl)],
        sss[(TLOOP - 1) % 3]).wait()

    plsc.subcore_barrier()

    for k in range(ROWS_PER_TILE // CP):
        r0 = row0 + k * CP
        pltpu.sync_copy(shared_agg.at[pl.ds(r0, CP)],
                        out_hbm.at[c, pl.ds(r0, CP)])

    @pl.when(is_tail)
    def _():
        pltpu.sync_copy(shared_agg.at[pl.ds(TAIL_R0, TAIL_ROWS)],
                        out_hbm.at[c, pl.ds(TAIL_R0, TAIL_ROWS)])


_sc_call = functools.partial(
    pl.kernel,
    out_type=jax.ShapeDtypeStruct((NC, N, D), jnp.float32),
    mesh=plsc.VectorSubcoreMesh(core_axis_name="c", subcore_axis_name="s"),
    scratch_types=[
        pltpu.VMEM_SHARED((N, D), jnp.float32),
        pltpu.VMEM((2, GR, SUB), jnp.int32),
        pltpu.VMEM((2, GR, SUB), jnp.int32),
        pltpu.VMEM((SUB, D), jnp.float32),
        pltpu.VMEM((SUB, D), jnp.float32),
        pltpu.VMEM((SUB, D), jnp.float32),
        pltpu.VMEM((SUB, D), jnp.float32),
        pltpu.VMEM((SUB, D), jnp.float32),
        pltpu.SemaphoreType.DMA,
        pltpu.SemaphoreType.DMA,
        pltpu.SemaphoreType.DMA,
        pltpu.SemaphoreType.DMA,
        pltpu.SemaphoreType.DMA,
        pltpu.SemaphoreType.DMA,
        pltpu.SemaphoreType.DMA,
        pltpu.SemaphoreType.DMA,
        pltpu.SemaphoreType.DMA,
    ],
)(_sc_aggregate)


def _mlp_bn(agg_ref, W1_ref, b1_ref, W2_ref, b2_ref, gamma_ref, beta_ref,
            out_ref):
    h0 = agg_ref[0] + agg_ref[1]
    h1 = jnp.maximum(
        jnp.dot(h0, W1_ref[...], preferred_element_type=jnp.float32)
        + b1_ref[...], 0.0)
    h2 = (jnp.dot(h1, W2_ref[...], preferred_element_type=jnp.float32)
          + b2_ref[...])
    mean = jnp.mean(h2, axis=0, keepdims=True)
    var = jnp.mean(h2 * h2, axis=0, keepdims=True) - mean * mean
    inv = jax.lax.rsqrt(var + 1e-5)
    out_ref[...] = (h2 - mean) * inv * gamma_ref[...] + beta_ref[...]


@jax.jit
def kernel(node_feats, edge_feats, W1, b1, W2, b2, gamma, beta, edge_index):
    pad = PAD_ROWS * SUB - E
    src = jnp.pad(edge_index[0], (0, pad)).reshape(PAD_ROWS, SUB)
    dst = jnp.pad(edge_index[1], (0, pad)).reshape(PAD_ROWS, SUB)
    agg = _sc_call(node_feats, edge_feats, src, dst)
    out = pl.pallas_call(
        _mlp_bn,
        out_shape=jax.ShapeDtypeStruct((N, D), jnp.float32),
    )(agg, W1, b1.reshape(1, D), W2, b2.reshape(1, D),
      gamma.reshape(1, D), beta.reshape(1, D))
    return out


# speedup vs baseline: 1.0067x; 1.0022x over previous
import functools

import jax
import jax.numpy as jnp
from jax import lax
from jax.experimental import pallas as pl
from jax.experimental.pallas import tpu as pltpu
from jax.experimental.pallas import tpu_sc as plsc

N = 10000
E = 320000
D = 128

NC = 2
NS = 16
NW = NC * NS
SUB = 32
NROW = E // SUB
TPT = 320
PAD_ROWS = TPT * NW
TLOOP = 324
GR = 8
ROWS_PER_TILE = 624
CP = 104
TAIL_R0 = NS * ROWS_PER_TILE
TAIL_ROWS = N - TAIL_R0
NLANE = D // 16


def _sc_aggregate(node_hbm, edge_hbm, src_hbm, dst_hbm, out_hbm,
                  shared_agg, sdb, ddb,
                  bg0, bg1, bg2, bg3, be0, be1,
                  sem_i, sg0, sg1, sg2, sg3, se0, se1,
                  ss0, ss1, ss2, ss3):
    c = lax.axis_index("c")
    s = lax.axis_index("s")
    wid = s * NC + c
    row_base = wid * TPT

    pltpu.sync_copy(src_hbm.at[pl.ds(row_base, GR)], sdb.at[0])
    pltpu.sync_copy(dst_hbm.at[pl.ds(row_base, GR)], ddb.at[0])
    pltpu.async_copy(src_hbm.at[pl.ds(row_base + GR, GR)], sdb.at[1], sem_i)
    pltpu.async_copy(dst_hbm.at[pl.ds(row_base + GR, GR)], ddb.at[1], sem_i)

    row0 = s * ROWS_PER_TILE
    is_tail = s == NS - 1

    @pl.when(c == 0)
    def _():
        for k in range(ROWS_PER_TILE // CP):
            r0 = row0 + k * CP
            pltpu.sync_copy(node_hbm.at[pl.ds(r0, CP)],
                            shared_agg.at[pl.ds(r0, CP)])

        @pl.when(is_tail)
        def _():
            pltpu.sync_copy(node_hbm.at[pl.ds(TAIL_R0, TAIL_ROWS)],
                            shared_agg.at[pl.ds(TAIL_R0, TAIL_ROWS)])

    @pl.when(c != 0)
    def _():
        def zrow(r, carry):
            for j in range(NLANE):
                bg0[r, pl.ds(j * 16, 16)] = jnp.zeros((16,), jnp.float32)
            return carry
        lax.fori_loop(0, SUB, zrow, 0)
        for k in range(ROWS_PER_TILE // CP):
            r0 = row0 + k * CP
            for b in range(0, CP, SUB):
                nrow = min(SUB, CP - b)
                pltpu.sync_copy(bg0.at[pl.ds(0, nrow)],
                                shared_agg.at[pl.ds(r0 + b, nrow)])

        @pl.when(is_tail)
        def _():
            pltpu.sync_copy(bg0.at[pl.ds(0, TAIL_ROWS)],
                            shared_agg.at[pl.ds(TAIL_R0, TAIL_ROWS)])

    plsc.subcore_barrier()

    bgs = (bg0, bg1, bg2, bg3)
    bes = (be0, be1)
    sgs = (sg0, sg1, sg2, sg3)
    ses = (se0, se1)
    sss = (ss0, ss1, ss2, ss3)

    def idx_slice(buf, t):
        return buf.at[(t // GR) % 2, t % GR]

    def issue_loads(t, bg, sg, be, se):
        pltpu.async_copy(node_hbm.at[idx_slice(sdb, t)], bg, sg)
        grow = row_base + t
        eoff = jnp.minimum(grow, NROW - 1) * SUB
        pltpu.async_copy(edge_hbm.at[pl.ds(eoff, SUB)], be, se)

    issue_loads(jnp.int32(0), bg0, sg0, be0, se0)
    issue_loads(jnp.int32(1), bg1, sg1, be1, se1)

    def ring_body(u, carry):
        for k in range(4):
            p2 = k % 2
            bg, sg = bgs[k], sgs[k]
            be, se = bes[p2], ses[p2]
            t = 4 * u + k
            grow = row_base + t
            pltpu.make_async_copy(
                node_hbm.at[idx_slice(sdb, t)], bg, sg).wait()
            eoff = jnp.minimum(grow, NROW - 1) * SUB
            pltpu.make_async_copy(
                edge_hbm.at[pl.ds(eoff, SUB)], be, se).wait()

            is_pad = jnp.logical_or(t >= TPT, grow >= NROW)

            @pl.when(jnp.logical_not(is_pad))
            def _():
                def rbody(r, rc):
                    for j in range(NLANE):
                        sl = pl.ds(j * 16, 16)
                        bg[r, sl] = jnp.maximum(bg[r, sl] + be[r, sl], 0.0)
                    return rc
                lax.fori_loop(0, SUB, rbody, 0)

            @pl.when(is_pad)
            def _():
                def zbody(r, rc):
                    for j in range(NLANE):
                        bg[r, pl.ds(j * 16, 16)] = jnp.zeros((16,),
                                                             jnp.float32)
                    return rc
                lax.fori_loop(0, SUB, zbody, 0)

            pltpu.async_copy(bg, shared_agg.at[idx_slice(ddb, t)],
                             sss[k], add=True)

            @pl.when(t >= 2)
            def _():
                tp = t - 2
                pltpu.make_async_copy(
                    bgs[(k + 2) % 4], shared_agg.at[idx_slice(ddb, tp)],
                    sss[(k + 2) % 4]).wait()

            @pl.when(jnp.logical_and(t % GR == 1, jnp.logical_and(
                t >= GR + 1, t < TLOOP - GR)))
            def _():
                g1 = t // GR + 1
                goff = jnp.minimum(row_base + GR * g1, PAD_ROWS - GR)
                pltpu.async_copy(src_hbm.at[pl.ds(goff, GR)],
                                 sdb.at[g1 % 2], sem_i)
                pltpu.async_copy(dst_hbm.at[pl.ds(goff, GR)],
                                 ddb.at[g1 % 2], sem_i)

            @pl.when(jnp.logical_and(t % GR == 6, t < TLOOP - 2))
            def _():
                g1 = t // GR + 1
                goff = jnp.minimum(row_base + GR * g1, PAD_ROWS - GR)
                pltpu.make_async_copy(
                    src_hbm.at[pl.ds(goff, GR)], sdb.at[g1 % 2],
                    sem_i).wait()
                pltpu.make_async_copy(
                    dst_hbm.at[pl.ds(goff, GR)], ddb.at[g1 % 2],
                    sem_i).wait()

            @pl.when(t + 2 < TLOOP)
            def _():
                issue_loads(t + 2, bgs[(k + 2) % 4], sgs[(k + 2) % 4],
                            be, se)
        return carry

    lax.fori_loop(0, TLOOP // 4, ring_body, 0)

    for toff in (TLOOP - 2, TLOOP - 1):
        tl = jnp.int32(toff)
        pltpu.make_async_copy(
            bgs[toff % 4], shared_agg.at[idx_slice(ddb, tl)],
            sss[toff % 4]).wait()

    plsc.subcore_barrier()

    for k in range(ROWS_PER_TILE // CP):
        r0 = row0 + k * CP
        pltpu.sync_copy(shared_agg.at[pl.ds(r0, CP)],
                        out_hbm.at[c, pl.ds(r0, CP)])

    @pl.when(is_tail)
    def _():
        pltpu.sync_copy(shared_agg.at[pl.ds(TAIL_R0, TAIL_ROWS)],
                        out_hbm.at[c, pl.ds(TAIL_R0, TAIL_ROWS)])


_sc_call = functools.partial(
    pl.kernel,
    out_type=jax.ShapeDtypeStruct((NC, N, D), jnp.float32),
    mesh=plsc.VectorSubcoreMesh(core_axis_name="c", subcore_axis_name="s"),
    scratch_types=[
        pltpu.VMEM_SHARED((N, D), jnp.float32),
        pltpu.VMEM((2, GR, SUB), jnp.int32),
        pltpu.VMEM((2, GR, SUB), jnp.int32),
        pltpu.VMEM((SUB, D), jnp.float32),
        pltpu.VMEM((SUB, D), jnp.float32),
        pltpu.VMEM((SUB, D), jnp.float32),
        pltpu.VMEM((SUB, D), jnp.float32),
        pltpu.VMEM((SUB, D), jnp.float32),
        pltpu.VMEM((SUB, D), jnp.float32),
        pltpu.SemaphoreType.DMA,
        pltpu.SemaphoreType.DMA,
        pltpu.SemaphoreType.DMA,
        pltpu.SemaphoreType.DMA,
        pltpu.SemaphoreType.DMA,
        pltpu.SemaphoreType.DMA,
        pltpu.SemaphoreType.DMA,
        pltpu.SemaphoreType.DMA,
        pltpu.SemaphoreType.DMA,
        pltpu.SemaphoreType.DMA,
        pltpu.SemaphoreType.DMA,
    ],
)(_sc_aggregate)


def _mlp_bn(agg_ref, W1_ref, b1_ref, W2_ref, b2_ref, gamma_ref, beta_ref,
            out_ref):
    h0 = agg_ref[0] + agg_ref[1]
    h1 = jnp.maximum(
        jnp.dot(h0, W1_ref[...], preferred_element_type=jnp.float32)
        + b1_ref[...], 0.0)
    h2 = (jnp.dot(h1, W2_ref[...], preferred_element_type=jnp.float32)
          + b2_ref[...])
    mean = jnp.mean(h2, axis=0, keepdims=True)
    var = jnp.mean(h2 * h2, axis=0, keepdims=True) - mean * mean
    inv = jax.lax.rsqrt(var + 1e-5)
    out_ref[...] = (h2 - mean) * inv * gamma_ref[...] + beta_ref[...]


@jax.jit
def kernel(node_feats, edge_feats, W1, b1, W2, b2, gamma, beta, edge_index):
    pad = PAD_ROWS * SUB - E
    src = jnp.pad(edge_index[0], (0, pad)).reshape(PAD_ROWS, SUB)
    dst = jnp.pad(edge_index[1], (0, pad)).reshape(PAD_ROWS, SUB)
    agg = _sc_call(node_feats, edge_feats, src, dst)
    out = pl.pallas_call(
        _mlp_bn,
        out_shape=jax.ShapeDtypeStruct((N, D), jnp.float32),
    )(agg, W1, b1.reshape(1, D), W2, b2.reshape(1, D),
      gamma.reshape(1, D), beta.reshape(1, D))
    return out


# speedup vs baseline: 1.0137x; 1.0069x over previous
import functools

import jax
import jax.numpy as jnp
from jax import lax
from jax.experimental import pallas as pl
from jax.experimental.pallas import tpu as pltpu
from jax.experimental.pallas import tpu_sc as plsc

N = 10000
E = 320000
D = 128

NC = 2
NS = 16
NW = NC * NS
SUB = 40
NROW = E // SUB
TPT = 256
PAD_ROWS = TPT * NW
GR = 8
ROWS_PER_TILE = 624
CP = 104
TAIL_R0 = NS * ROWS_PER_TILE
TAIL_ROWS = N - TAIL_R0
NLANE = D // 16


def _sc_aggregate(node_hbm, edge_hbm, src_hbm, dst_hbm, out_hbm,
                  shared_agg, sdb, ddb,
                  bg0, bg1, be0, be1, bm0, bm1,
                  sem_i, sg0, sg1, se0, se1, ss0, ss1):
    c = lax.axis_index("c")
    s = lax.axis_index("s")
    wid = s * NC + c
    row_base = wid * TPT

    pltpu.sync_copy(src_hbm.at[pl.ds(row_base, GR)], sdb.at[0])
    pltpu.sync_copy(dst_hbm.at[pl.ds(row_base, GR)], ddb.at[0])
    pltpu.async_copy(src_hbm.at[pl.ds(row_base + GR, GR)], sdb.at[1], sem_i)
    pltpu.async_copy(dst_hbm.at[pl.ds(row_base + GR, GR)], ddb.at[1], sem_i)

    row0 = s * ROWS_PER_TILE
    is_tail = s == NS - 1

    @pl.when(c == 0)
    def _():
        for k in range(ROWS_PER_TILE // CP):
            r0 = row0 + k * CP
            pltpu.sync_copy(node_hbm.at[pl.ds(r0, CP)],
                            shared_agg.at[pl.ds(r0, CP)])

        @pl.when(is_tail)
        def _():
            pltpu.sync_copy(node_hbm.at[pl.ds(TAIL_R0, TAIL_ROWS)],
                            shared_agg.at[pl.ds(TAIL_R0, TAIL_ROWS)])

    @pl.when(c != 0)
    def _():
        def zrow(r, carry):
            for j in range(NLANE):
                bg0[r, pl.ds(j * 16, 16)] = jnp.zeros((16,), jnp.float32)
            return carry
        lax.fori_loop(0, SUB, zrow, 0)
        for k in range(ROWS_PER_TILE // CP):
            r0 = row0 + k * CP
            for b in range(0, CP, SUB):
                nrow = min(SUB, CP - b)
                pltpu.sync_copy(bg0.at[pl.ds(0, nrow)],
                                shared_agg.at[pl.ds(r0 + b, nrow)])

        @pl.when(is_tail)
        def _():
            pltpu.sync_copy(bg0.at[pl.ds(0, TAIL_ROWS)],
                            shared_agg.at[pl.ds(TAIL_R0, TAIL_ROWS)])

    plsc.subcore_barrier()

    bufs = ((bg0, be0, bm0, sg0, se0, ss0),
            (bg1, be1, bm1, sg1, se1, ss1))

    def idx_slice(buf, t):
        return buf.at[(t // GR) % 2, t % GR]

    def issue_loads(t, bg, sg, be, se):
        pltpu.async_copy(node_hbm.at[idx_slice(sdb, t)], bg, sg)
        grow = row_base + t
        eoff = jnp.minimum(grow, NROW - 1) * SUB
        pltpu.async_copy(edge_hbm.at[pl.ds(eoff, SUB)], be, se)

    issue_loads(jnp.int32(0), bg0, sg0, be0, se0)
    issue_loads(jnp.int32(1), bg1, sg1, be1, se1)

    def pair_body(u, carry):
        for p in range(2):
            bg, be, bm, sg, se, ss = bufs[p]
            t = 2 * u + p
            grow = row_base + t
            pltpu.make_async_copy(
                node_hbm.at[idx_slice(sdb, t)], bg, sg).wait()
            eoff = jnp.minimum(grow, NROW - 1) * SUB
            pltpu.make_async_copy(
                edge_hbm.at[pl.ds(eoff, SUB)], be, se).wait()

            @pl.when(u >= 1)
            def _():
                tp = t - 2
                pltpu.make_async_copy(
                    bm, shared_agg.at[idx_slice(ddb, tp)], ss).wait()

            is_pad = grow >= NROW

            @pl.when(jnp.logical_not(is_pad))
            def _():
                def rbody(r, rc):
                    for j in range(NLANE):
                        sl = pl.ds(j * 16, 16)
                        bm[r, sl] = jnp.maximum(bg[r, sl] + be[r, sl], 0.0)
                    return rc
                lax.fori_loop(0, SUB, rbody, 0)

            @pl.when(is_pad)
            def _():
                def zbody(r, rc):
                    for j in range(NLANE):
                        bm[r, pl.ds(j * 16, 16)] = jnp.zeros((16,),
                                                             jnp.float32)
                    return rc
                lax.fori_loop(0, SUB, zbody, 0)

            pltpu.async_copy(bm, shared_agg.at[idx_slice(ddb, t)],
                             ss, add=True)

            @pl.when(jnp.logical_and(t % GR == 1, jnp.logical_and(
                t >= GR + 1, t < TPT - 2 * GR + 2)))
            def _():
                g1 = t // GR + 1
                goff = row_base + GR * g1
                pltpu.async_copy(src_hbm.at[pl.ds(goff, GR)],
                                 sdb.at[g1 % 2], sem_i)
                pltpu.async_copy(dst_hbm.at[pl.ds(goff, GR)],
                                 ddb.at[g1 % 2], sem_i)

            @pl.when(jnp.logical_and(t % GR == 6, t < TPT - GR - 1))
            def _():
                g1 = t // GR + 1
                goff = row_base + GR * g1
                pltpu.make_async_copy(
                    src_hbm.at[pl.ds(goff, GR)], sdb.at[g1 % 2],
                    sem_i).wait()
                pltpu.make_async_copy(
                    dst_hbm.at[pl.ds(goff, GR)], ddb.at[g1 % 2],
                    sem_i).wait()

            @pl.when(t + 2 < TPT)
            def _():
                issue_loads(t + 2, bg, sg, be, se)
        return carry

    lax.fori_loop(0, TPT // 2, pair_body, 0)

    for toff in (TPT - 2, TPT - 1):
        tl = jnp.int32(toff)
        bg, be, bm, sg, se, ss = bufs[toff % 2]
        pltpu.make_async_copy(
            bm, shared_agg.at[idx_slice(ddb, tl)], ss).wait()

    plsc.subcore_barrier()

    for k in range(ROWS_PER_TILE // CP):
        r0 = row0 + k * CP
        pltpu.sync_copy(shared_agg.at[pl.ds(r0, CP)],
                        out_hbm.at[c, pl.ds(r0, CP)])

    @pl.when(is_tail)
    def _():
        pltpu.sync_copy(shared_agg.at[pl.ds(TAIL_R0, TAIL_ROWS)],
                        out_hbm.at[c, pl.ds(TAIL_R0, TAIL_ROWS)])


_sc_call = functools.partial(
    pl.kernel,
    out_type=jax.ShapeDtypeStruct((NC, N, D), jnp.float32),
    mesh=plsc.VectorSubcoreMesh(core_axis_name="c", subcore_axis_name="s"),
    scratch_types=[
        pltpu.VMEM_SHARED((N, D), jnp.float32),
        pltpu.VMEM((2, GR, SUB), jnp.int32),
        pltpu.VMEM((2, GR, SUB), jnp.int32),
        pltpu.VMEM((SUB, D), jnp.float32),
        pltpu.VMEM((SUB, D), jnp.float32),
        pltpu.VMEM((SUB, D), jnp.float32),
        pltpu.VMEM((SUB, D), jnp.float32),
        pltpu.VMEM((SUB, D), jnp.float32),
        pltpu.VMEM((SUB, D), jnp.float32),
        pltpu.SemaphoreType.DMA,
        pltpu.SemaphoreType.DMA,
        pltpu.SemaphoreType.DMA,
        pltpu.SemaphoreType.DMA,
        pltpu.SemaphoreType.DMA,
        pltpu.SemaphoreType.DMA,
        pltpu.SemaphoreType.DMA,
    ],
)(_sc_aggregate)


def _mlp_bn(agg_ref, W1_ref, b1_ref, W2_ref, b2_ref, gamma_ref, beta_ref,
            out_ref):
    h0 = agg_ref[0] + agg_ref[1]
    h1 = jnp.maximum(
        jnp.dot(h0, W1_ref[...], preferred_element_type=jnp.float32)
        + b1_ref[...], 0.0)
    h2 = (jnp.dot(h1, W2_ref[...], preferred_element_type=jnp.float32)
          + b2_ref[...])
    mean = jnp.mean(h2, axis=0, keepdims=True)
    var = jnp.mean(h2 * h2, axis=0, keepdims=True) - mean * mean
    inv = jax.lax.rsqrt(var + 1e-5)
    out_ref[...] = (h2 - mean) * inv * gamma_ref[...] + beta_ref[...]


@jax.jit
def kernel(node_feats, edge_feats, W1, b1, W2, b2, gamma, beta, edge_index):
    pad = PAD_ROWS * SUB - E
    src = jnp.pad(edge_index[0], (0, pad)).reshape(PAD_ROWS, SUB)
    dst = jnp.pad(edge_index[1], (0, pad)).reshape(PAD_ROWS, SUB)
    agg = _sc_call(node_feats, edge_feats, src, dst)
    out = pl.pallas_call(
        _mlp_bn,
        out_shape=jax.ShapeDtypeStruct((N, D), jnp.float32),
    )(agg, W1, b1.reshape(1, D), W2, b2.reshape(1, D),
      gamma.reshape(1, D), beta.reshape(1, D))
    return out


# speedup vs baseline: 2.1988x; 2.1692x over previous
import functools

import jax
import jax.numpy as jnp
from jax import lax
from jax.experimental import pallas as pl
from jax.experimental.pallas import tpu as pltpu
from jax.experimental.pallas import tpu_sc as plsc

N = 10000
E = 320000
D = 128

NC = 2
NS = 16
NW = NC * NS
RW = 128
SUB = 32
QN = RW // SUB
NROW = E // RW
SC_ROWS = 8
NSUPER = -(-NROW // SC_ROWS)
MAX_SUPER = -(-NSUPER // NW)
PAD_ROWS = MAX_SUPER * NW * SC_ROWS
TROWS = MAX_SUPER * SC_ROWS
ROWS_PER_TILE = 624
CP = 104
TAIL_R0 = NS * ROWS_PER_TILE
TAIL_ROWS = N - TAIL_R0
NLANE = D // 16


def _sc_aggregate(node_hbm, edge_hbm, src_hbm, dst_hbm, out_hbm,
                  shared_agg, idx_s, idx_d, sid0, sid1,
                  buf_g0, buf_g1, buf_e0, buf_e1, buf_m0, buf_m1,
                  sem_i, sem_g0, sem_g1, sem_e0, sem_e1, sem_s0, sem_s1):
    c = lax.axis_index("c")
    s = lax.axis_index("s")
    wid = s * NC + c

    nsc = NSUPER // NW + jnp.where(wid < (NSUPER % NW), 1, 0)
    nch = nsc * SC_ROWS * QN

    idx_cps = []
    for i in range(MAX_SUPER):
        r0 = SC_ROWS * (wid + NW * i)
        idx_cps.append(pltpu.async_copy(
            src_hbm.at[pl.ds(r0, SC_ROWS)],
            idx_s.at[pl.ds(SC_ROWS * i, SC_ROWS)], sem_i))
        idx_cps.append(pltpu.async_copy(
            dst_hbm.at[pl.ds(r0, SC_ROWS)],
            idx_d.at[pl.ds(SC_ROWS * i, SC_ROWS)], sem_i))

    row0 = s * ROWS_PER_TILE
    is_tail = s == NS - 1

    @pl.when(c == 0)
    def _():
        for k in range(ROWS_PER_TILE // CP):
            r0 = row0 + k * CP
            pltpu.sync_copy(node_hbm.at[pl.ds(r0, CP)],
                            shared_agg.at[pl.ds(r0, CP)])

        @pl.when(is_tail)
        def _():
            pltpu.sync_copy(node_hbm.at[pl.ds(TAIL_R0, TAIL_ROWS)],
                            shared_agg.at[pl.ds(TAIL_R0, TAIL_ROWS)])

    @pl.when(c != 0)
    def _():
        def zrow(r, carry):
            for j in range(NLANE):
                buf_g0[r, pl.ds(j * 16, 16)] = jnp.zeros((16,), jnp.float32)
            return carry
        lax.fori_loop(0, SUB, zrow, 0)
        for k in range(ROWS_PER_TILE // CP):
            r0 = row0 + k * CP
            for b in range(0, CP, SUB):
                nrow = min(SUB, CP - b)
                pltpu.sync_copy(buf_g0.at[pl.ds(0, nrow)],
                                shared_agg.at[pl.ds(r0 + b, nrow)])

        @pl.when(is_tail)
        def _():
            pltpu.sync_copy(buf_g0.at[pl.ds(0, TAIL_ROWS)],
                            shared_agg.at[pl.ds(TAIL_R0, TAIL_ROWS)])

    del idx_cps
    pltpu.make_async_copy(src_hbm.at[pl.ds(0, TROWS)], idx_s, sem_i).wait()
    pltpu.make_async_copy(src_hbm.at[pl.ds(0, TROWS)], idx_d, sem_i).wait()

    plsc.subcore_barrier()

    bufs = ((buf_g0, buf_e0, buf_m0, sid0, sem_g0, sem_e0, sem_s0),
            (buf_g1, buf_e1, buf_m1, sid1, sem_g1, sem_e1, sem_s1))

    def locate(t):
        lrow = t // QN
        q = t % QN
        grow = SC_ROWS * (wid + NW * (lrow // SC_ROWS)) + lrow % SC_ROWS
        return lrow, q, grow

    def issue_loads(t, bg, be, sg, se):
        lrow, q, grow = locate(t)
        pltpu.async_copy(
            node_hbm.at[idx_s.at[lrow, pl.ds(SUB * q, SUB)]], bg, sg)
        eoff = jnp.where(grow < NROW, grow, 0) * RW + SUB * q
        pltpu.async_copy(edge_hbm.at[pl.ds(eoff, SUB)], be, se)

    issue_loads(jnp.int32(0), buf_g0, buf_e0, sem_g0, sem_e0)
    issue_loads(jnp.int32(1), buf_g1, buf_e1, sem_g1, sem_e1)

    def pair_body(u, carry):
        for p in range(2):
            bg, be, bm, sid, sg, se, ss = bufs[p]
            t = 2 * u + p
            lrow, q, grow = locate(t)
            pltpu.make_async_copy(
                node_hbm.at[idx_s.at[lrow, pl.ds(SUB * q, SUB)]],
                bg, sg).wait()
            eoff = jnp.where(grow < NROW, grow, 0) * RW + SUB * q
            pltpu.make_async_copy(
                edge_hbm.at[pl.ds(eoff, SUB)], be, se).wait()

            @pl.when(u >= 1)
            def _():
                pltpu.make_async_copy(bm, shared_agg.at[sid], ss).wait()

            is_pad = grow >= NROW

            @pl.when(jnp.logical_not(is_pad))
            def _():
                def rbody(r, rc):
                    for j in range(NLANE):
                        sl = pl.ds(j * 16, 16)
                        bm[r, sl] = jnp.maximum(bg[r, sl] + be[r, sl], 0.0)
                    return rc
                lax.fori_loop(0, SUB, rbody, 0)

            @pl.when(is_pad)
            def _():
                def zbody(r, rc):
                    for j in range(NLANE):
                        bm[r, pl.ds(j * 16, 16)] = jnp.zeros((16,),
                                                             jnp.float32)
                    return rc
                lax.fori_loop(0, SUB, zbody, 0)

            sid[pl.ds(0, 16)] = idx_d[lrow, pl.ds(SUB * q, 16)]
            sid[pl.ds(16, 16)] = idx_d[lrow, pl.ds(SUB * q + 16, 16)]

            pltpu.async_copy(bm, shared_agg.at[sid], ss, add=True)

            @pl.when(t + 2 < nch)
            def _():
                issue_loads(t + 2, bg, be, sg, se)
        return carry

    lax.fori_loop(0, nch // 2, pair_body, 0)

    for p in range(2):
        bg, be, bm, sid, sg, se, ss = bufs[p]
        pltpu.make_async_copy(bm, shared_agg.at[sid], ss).wait()

    plsc.subcore_barrier()

    for k in range(ROWS_PER_TILE // CP):
        r0 = row0 + k * CP
        pltpu.sync_copy(shared_agg.at[pl.ds(r0, CP)],
                        out_hbm.at[c, pl.ds(r0, CP)])

    @pl.when(is_tail)
    def _():
        pltpu.sync_copy(shared_agg.at[pl.ds(TAIL_R0, TAIL_ROWS)],
                        out_hbm.at[c, pl.ds(TAIL_R0, TAIL_ROWS)])


_sc_call = functools.partial(
    pl.kernel,
    out_type=jax.ShapeDtypeStruct((NC, N, D), jnp.float32),
    mesh=plsc.VectorSubcoreMesh(core_axis_name="c", subcore_axis_name="s"),
    scratch_types=[
        pltpu.VMEM_SHARED((N, D), jnp.float32),
        pltpu.VMEM((TROWS, RW), jnp.int32),
        pltpu.VMEM((TROWS, RW), jnp.int32),
        pltpu.VMEM((SUB,), jnp.int32),
        pltpu.VMEM((SUB,), jnp.int32),
        pltpu.VMEM((SUB, D), jnp.float32),
        pltpu.VMEM((SUB, D), jnp.float32),
        pltpu.VMEM((SUB, D), jnp.float32),
        pltpu.VMEM((SUB, D), jnp.float32),
        pltpu.VMEM((SUB, D), jnp.float32),
        pltpu.VMEM((SUB, D), jnp.float32),
        pltpu.SemaphoreType.DMA,
        pltpu.SemaphoreType.DMA,
        pltpu.SemaphoreType.DMA,
        pltpu.SemaphoreType.DMA,
        pltpu.SemaphoreType.DMA,
        pltpu.SemaphoreType.DMA,
        pltpu.SemaphoreType.DMA,
    ],
)(_sc_aggregate)


def _mlp_bn(agg_ref, W1_ref, b1_ref, W2_ref, b2_ref, gamma_ref, beta_ref,
            out_ref):
    h0 = agg_ref[0] + agg_ref[1]
    h1 = jnp.maximum(
        jnp.dot(h0, W1_ref[...], preferred_element_type=jnp.float32)
        + b1_ref[...], 0.0)
    h2 = (jnp.dot(h1, W2_ref[...], preferred_element_type=jnp.float32)
          + b2_ref[...])
    mean = jnp.mean(h2, axis=0, keepdims=True)
    var = jnp.mean(h2 * h2, axis=0, keepdims=True) - mean * mean
    inv = jax.lax.rsqrt(var + 1e-5)
    out_ref[...] = (h2 - mean) * inv * gamma_ref[...] + beta_ref[...]


@jax.jit
def kernel(node_feats, edge_feats, W1, b1, W2, b2, gamma, beta, edge_index):
    pad = PAD_ROWS * RW - E
    src = jnp.pad(edge_index[0], (0, pad)).reshape(PAD_ROWS, RW)
    dst = jnp.pad(edge_index[1], (0, pad)).reshape(PAD_ROWS, RW)
    agg = _sc_call(node_feats, edge_feats, src, dst)
    out = pl.pallas_call(
        _mlp_bn,
        out_shape=jax.ShapeDtypeStruct((N, D), jnp.float32),
    )(agg, W1, b1.reshape(1, D), W2, b2.reshape(1, D),
      gamma.reshape(1, D), beta.reshape(1, D))
    return out
